# baseline (speedup 1.0000x reference)
import functools

import jax
import jax.numpy as jnp
from jax import lax
from jax.experimental import pallas as pl
from jax.experimental.pallas import tpu as pltpu
from jax.experimental.pallas import tpu_sc as plsc

NTOKEN = 100000
NTOKEN_C = 1000
EMB_DIM = 300
C_EMB_DIM = 64
EMB_PAD = 384
C_EMB_PAD = 128

B0, B1 = 4096, 50
NC, NS = 2, 16
NW = NC * NS
IBLK = B0 // NW
NG = IBLK // 16


def _make_embed_kernel():
    mesh = plsc.VectorSubcoreMesh(core_axis_name="c", subcore_axis_name="s")

    @functools.partial(
        pl.kernel,
        mesh=mesh,
        out_type=(
            jax.ShapeDtypeStruct((B1, EMB_DIM, B0), jnp.float32),
            jax.ShapeDtypeStruct((B1, C_EMB_DIM, B0), jnp.float32),
        ),
        scratch_types=[
            pltpu.VMEM((B1, IBLK), jnp.int32),
            pltpu.VMEM((B1, IBLK), jnp.int32),
            pltpu.VMEM((IBLK, EMB_PAD), jnp.float32),
            pltpu.VMEM((EMB_DIM, IBLK), jnp.float32),
            pltpu.VMEM((IBLK, C_EMB_PAD), jnp.float32),
            pltpu.VMEM((C_EMB_DIM, IBLK), jnp.float32),
            pltpu.SemaphoreType.DMA,
            pltpu.SemaphoreType.DMA,
            pltpu.SemaphoreType.DMA,
            pltpu.SemaphoreType.DMA,
        ],
        compiler_params=pltpu.CompilerParams(needs_layout_passes=False),
    )
    def embed_kernel(xt_hbm, xct_hbm, emb_hbm, cemb_hbm, out_hbm, outc_hbm,
                     idx_v, idxc_v, rows_v, rowst_v, crows_v, crowst_v,
                     gsem, gsem2, wsem, wsem2):
        wid = lax.axis_index("s") * NC + lax.axis_index("c")
        col0 = wid * IBLK
        pltpu.sync_copy(xt_hbm.at[:, pl.ds(col0, IBLK)], idx_v)
        pltpu.sync_copy(xct_hbm.at[:, pl.ds(col0, IBLK)], idxc_v)

        tok16 = [lax.iota(jnp.int32, 16) + 16 * g for g in range(NG)]

        def xpose(src, dst, d_hi):
            @plsc.parallel_loop(0, d_hi, 1, unroll=8)
            def _(d):
                dvec = jnp.full((16,), d, jnp.int32)
                for g in range(NG):
                    vals = plsc.load_gather(src, [tok16[g], dvec])
                    dst[d, pl.ds(16 * g, 16)] = vals

        def body(j, carry):
            out_slc = out_hbm.at[j, :, pl.ds(col0, IBLK)]
            outc_slc = outc_hbm.at[j, :, pl.ds(col0, IBLK)]
            pltpu.async_copy(emb_hbm.at[idx_v.at[j]], rows_v, gsem)
            pltpu.async_copy(cemb_hbm.at[idxc_v.at[j]], crows_v, gsem2)
            pltpu.make_async_copy(emb_hbm.at[idx_v.at[j]], rows_v, gsem).wait()
            xpose(rows_v, rowst_v, EMB_DIM)
            pltpu.sync_copy(rowst_v, out_slc)
            pltpu.make_async_copy(cemb_hbm.at[idxc_v.at[j]], crows_v,
                                  gsem2).wait()
            xpose(crows_v, crowst_v, C_EMB_DIM)
            pltpu.sync_copy(crowst_v, outc_slc)
            return carry

        lax.fori_loop(0, B1, body, 0)

    return embed_kernel


_embed = _make_embed_kernel()


def kernel(x, x_c, emb_W, c_emb_W):
    xt = x.T.astype(jnp.int32)
    xct = x_c.T.astype(jnp.int32)
    emb_p = jnp.pad(emb_W, ((0, 0), (0, EMB_PAD - EMB_DIM)))
    cemb_p = jnp.pad(c_emb_W, ((0, 0), (0, C_EMB_PAD - C_EMB_DIM)))
    out3, outc3 = _embed(xt, xct, emb_p, cemb_p)
    return (jnp.transpose(out3, (2, 0, 1)), jnp.transpose(outc3, (2, 0, 1)))
